# baseline (speedup 1.0000x reference)
import functools

import jax
import jax.numpy as jnp
from jax import lax
from jax.experimental import pallas as pl
from jax.experimental.pallas import tpu as pltpu
from jax.experimental.pallas import tpu_sc as plsc

_REDUCTION_RATE = 0.9
_L = 16


def _mo8(x):
    return pl.multiple_of(x, 8)


def _build(B, N, D, H):
    M = max(1, int(N * (1.0 - _REDUCTION_RATE)))
    n_ch = -(-M // _L)
    m_pad = -(-M // 8) * 8
    tail_off = m_pad - _L
    rows_total = B * H
    rows_per_tile = rows_total // _L
    nvec = N // _L
    cols_per_tile = N // _L
    mesh = plsc.VectorSubcoreMesh(core_axis_name="c", subcore_axis_name="s")

    @functools.partial(
        pl.kernel,
        out_type=jax.ShapeDtypeStruct((B, M, D), jnp.float32),
        mesh=mesh,
        scratch_types=[
            pltpu.VMEM((rows_per_tile, N), jnp.float32),
            pltpu.VMEM((N,), jnp.float32),
            pltpu.VMEM((_L, cols_per_tile), jnp.float32),
            pltpu.VMEM((cols_per_tile,), jnp.float32),
            pltpu.VMEM((N,), jnp.float32),
            pltpu.VMEM((N,), jnp.int32),
            pltpu.VMEM((N,), jnp.int32),
            pltpu.VMEM((128,), jnp.int32),
            pltpu.VMEM((_L,), jnp.int32),
            pltpu.VMEM((_L, D), jnp.float32),
            pltpu.VMEM_SHARED((_L * N,), jnp.float32),
            pltpu.VMEM_SHARED((N,), jnp.float32),
            pltpu.VMEM_SHARED((m_pad,), jnp.int32),
            pltpu.SemaphoreType.DMA,
        ],
        compiler_params=pltpu.CompilerParams(needs_layout_passes=False),
    )
    def k(cls_hbm, vt_hbm, out_hbm, row4_v, part_v, sub_v, loc_v, scores_v,
          keys_v, cand_v, kept_v, idx16_v, rows_v, shared_part,
          shared_scores, shared_idx, sem):
        cid = lax.axis_index("c")
        sid = lax.axis_index("s")

        copies = []
        for j in range(rows_per_tile):
            r = sid * rows_per_tile + j
            copies.append(
                pltpu.async_copy(cls_hbm.at[pl.ds(_mo8(r * N), N)],
                                 row4_v.at[j], sem))
        for c in copies:
            c.wait()

        def _sum_chunk(c, _):
            sl = pl.ds(c * _L, _L)
            acc = row4_v[0, sl]
            for j in range(1, rows_per_tile):
                acc = acc + row4_v[j, sl]
            part_v[sl] = acc
            return 0

        lax.fori_loop(0, nvec, _sum_chunk, 0)
        pltpu.sync_copy(part_v, shared_part.at[pl.ds(_mo8(sid * N), N)])
        plsc.subcore_barrier()

        col0 = sid * cols_per_tile
        for t in range(_L):
            pltpu.sync_copy(
                shared_part.at[pl.ds(_mo8(t * N + col0), cols_per_tile)],
                sub_v.at[t])
        for c in range(cols_per_tile // _L):
            sl = pl.ds(c * _L, _L)
            acc = sub_v[0, sl]
            for t in range(1, _L):
                acc = acc + sub_v[t, sl]
            loc_v[sl] = acc
        pltpu.sync_copy(loc_v,
                        shared_scores.at[pl.ds(_mo8(col0), cols_per_tile)])
        plsc.subcore_barrier()

        @pl.when(sid == 0)
        def _select():
            pltpu.sync_copy(shared_scores, scores_v)

            def _key_chunk(c, acc):
                sl = pl.ds(c * _L, _L)
                iv = lax.bitcast_convert_type(scores_v[sl], jnp.int32)
                key = jnp.where(iv >= 0, iv, iv ^ jnp.int32(0x7FFFFFFF))
                keys_v[sl] = key
                return acc + jnp.where(key >= 0, 1, 0).astype(jnp.int32)

            cnt_pos = jnp.sum(
                lax.fori_loop(0, nvec, _key_chunk,
                              jnp.zeros((_L,), jnp.int32)))

            def count_ge(thresh):
                acc = jnp.zeros((_L,), jnp.int32)
                for c in range(nvec):
                    kv = keys_v[pl.ds(c * _L, _L)]
                    acc = acc + jnp.where(kv >= thresh, 1, 0).astype(jnp.int32)
                return jnp.sum(acc)

            pos_bucket = cnt_pos >= M
            lo0 = jnp.where(pos_bucket, jnp.int32(0), jnp.int32(-(2**31)))
            hi0 = jnp.where(pos_bucket, jnp.int32(2**31 - 1), jnp.int32(-1))

            def bis(_, carry):
                lo, hi = carry
                mid = lo + 1 + ((hi - lo - 1) >> 1)
                ge = count_ge(mid) >= M
                return (jnp.where(ge, mid, lo), jnp.where(ge, hi, mid - 1))

            lo4, hi4 = lax.fori_loop(0, 4, bis, (lo0, hi0))

            def _fill(c, _):
                cand_v[pl.ds(c * _L, _L)] = jnp.full((_L,), -(2**31),
                                                     jnp.int32)
                return 0

            lax.fori_loop(0, nvec, _fill, 0)

            na = jnp.int32(0)
            acc_hi = jnp.zeros((_L,), jnp.int32)
            for c in range(nvec):
                kv = keys_v[pl.ds(c * _L, _L)]
                acc_hi = acc_hi + jnp.where(kv > hi4, 1, 0).astype(jnp.int32)
                m_in = (kv >= lo4) & (kv <= hi4)
                plsc.store_compressed(cand_v.at[pl.ds(na, _L)], kv,
                                      mask=m_in)
                na = na + jnp.sum(m_in.astype(jnp.int32))
            cnt_hi = jnp.sum(acc_hi)
            M2 = M - cnt_hi
            nch = (na + _L - 1) >> 4

            def count_ge_cand(thresh):
                def body(c, acc):
                    kc = cand_v[pl.ds(c * _L, _L)]
                    return acc + jnp.where(kc >= thresh, 1, 0
                                           ).astype(jnp.int32)
                return jnp.sum(lax.fori_loop(0, nch, body,
                                             jnp.zeros((_L,), jnp.int32)))

            def bis2(_, carry):
                lo, hi = carry
                mid = lo + 1 + ((hi - lo - 1) >> 1)
                ge = count_ge_cand(mid) >= M2
                return (jnp.where(ge, mid, lo), jnp.where(ge, hi, mid - 1))

            v_key, _ = lax.fori_loop(0, 27, bis2, (lo4, hi4))

            def _gt_body(c, acc):
                kc = cand_v[pl.ds(c * _L, _L)]
                return acc + jnp.where(kc > v_key, 1, 0).astype(jnp.int32)

            need_eq = M2 - jnp.sum(lax.fori_loop(0, nch, _gt_body,
                                                 jnp.zeros((_L,), jnp.int32)))

            for c in range(128 // _L):
                kept_v[pl.ds(c * _L, _L)] = jnp.zeros((_L,), jnp.int32)

            def _compact(c, carry):
                pos, eq_used = carry
                kv = keys_v[pl.ds(c * _L, _L)]
                idxs = lax.iota(jnp.int32, _L) + c * _L
                gt = kv > v_key
                eq = kv == v_key
                eq_i = eq.astype(jnp.int32)
                pref = plsc.cumsum(eq_i)
                ties_before = eq_used + pref - eq_i
                keep = gt | (eq & (ties_before < need_eq))
                plsc.store_compressed(kept_v.at[pl.ds(pos, _L)], idxs,
                                      mask=keep)
                return (pos + jnp.sum(keep.astype(jnp.int32)),
                        eq_used + jnp.sum(eq_i))

            lax.fori_loop(0, nvec, _compact, (jnp.int32(0), jnp.int32(0)))
            pltpu.sync_copy(kept_v.at[pl.ds(0, m_pad)], shared_idx)

        plsc.subcore_barrier()

        bpc = B // 2

        @pl.when(sid < bpc * n_ch)
        def _gather():
            b = cid * bpc + sid // n_ch
            ch = lax.rem(sid, n_ch)
            row_off = _mo8(jnp.where(ch == n_ch - 1, tail_off, ch * _L))
            pltpu.sync_copy(shared_idx.at[pl.ds(row_off, _L)], idx16_v)
            gidx = idx16_v[...] + b * N
            pltpu.async_copy(vt_hbm.at[gidx], rows_v, sem).wait()
            pltpu.sync_copy(rows_v, out_hbm.at[b, pl.ds(row_off, _L)])

    return k


def kernel(visual_tokens, attention_weights):
    B, N, D = visual_tokens.shape
    H = attention_weights.shape[1]
    cls_rows = attention_weights[:, :, 0, 1:].reshape(B * H * N)
    vt2 = visual_tokens.reshape(B * N, D)
    return _build(B, N, D, H)(cls_rows, vt2)


# speedup vs baseline: 1.2877x; 1.2877x over previous
import functools

import jax
import jax.numpy as jnp
from jax import lax
from jax.experimental import pallas as pl
from jax.experimental.pallas import tpu as pltpu
from jax.experimental.pallas import tpu_sc as plsc

_REDUCTION_RATE = 0.9
_L = 16


def _mo8(x):
    return pl.multiple_of(x, 8)


def _build(B, N, D, H):
    M = max(1, int(N * (1.0 - _REDUCTION_RATE)))
    n_ch = -(-M // _L)
    m_pad = -(-M // 8) * 8
    tail_off = m_pad - _L
    rows_total = B * H
    rows_per_tile = rows_total // _L
    nvec = N // _L
    cols_per_tile = N // _L
    mesh = plsc.VectorSubcoreMesh(core_axis_name="c", subcore_axis_name="s")

    @functools.partial(
        pl.kernel,
        out_type=jax.ShapeDtypeStruct((B, M, D), jnp.float32),
        mesh=mesh,
        scratch_types=[
            pltpu.VMEM((rows_per_tile, N), jnp.float32),
            pltpu.VMEM((N,), jnp.float32),
            pltpu.VMEM((_L, cols_per_tile), jnp.float32),
            pltpu.VMEM((cols_per_tile,), jnp.float32),
            pltpu.VMEM((N,), jnp.float32),
            pltpu.VMEM((N,), jnp.int32),
            pltpu.VMEM((128,), jnp.int32),
            pltpu.VMEM((_L,), jnp.int32),
            pltpu.VMEM((_L, D), jnp.float32),
            pltpu.VMEM_SHARED((_L * N,), jnp.float32),
            pltpu.VMEM_SHARED((N,), jnp.float32),
            pltpu.VMEM_SHARED((m_pad,), jnp.int32),
            pltpu.SemaphoreType.DMA,
        ],
        compiler_params=pltpu.CompilerParams(needs_layout_passes=False),
    )
    def k(cls_hbm, vt_hbm, out_hbm, row4_v, part_v, sub_v, loc_v, scores_v,
          keys_v, kept_v, idx16_v, rows_v, shared_part, shared_scores,
          shared_idx, sem):
        cid = lax.axis_index("c")
        sid = lax.axis_index("s")

        copies = []
        for j in range(rows_per_tile):
            r = sid * rows_per_tile + j
            copies.append(
                pltpu.async_copy(cls_hbm.at[pl.ds(_mo8(r * N), N)],
                                 row4_v.at[j], sem))
        for c in copies:
            c.wait()

        def _sum_chunk(c, _):
            sl = pl.ds(c * _L, _L)
            acc = row4_v[0, sl]
            for j in range(1, rows_per_tile):
                acc = acc + row4_v[j, sl]
            part_v[sl] = acc
            return 0

        lax.fori_loop(0, nvec, _sum_chunk, 0)
        pltpu.sync_copy(part_v, shared_part.at[pl.ds(_mo8(sid * N), N)])
        plsc.subcore_barrier()

        col0 = sid * cols_per_tile
        for t in range(_L):
            pltpu.sync_copy(
                shared_part.at[pl.ds(_mo8(t * N + col0), cols_per_tile)],
                sub_v.at[t])
        for c in range(cols_per_tile // _L):
            sl = pl.ds(c * _L, _L)
            acc = sub_v[0, sl]
            for t in range(1, _L):
                acc = acc + sub_v[t, sl]
            loc_v[sl] = acc
        pltpu.sync_copy(loc_v,
                        shared_scores.at[pl.ds(_mo8(col0), cols_per_tile)])
        plsc.subcore_barrier()

        @pl.when(sid == 0)
        def _select():
            pltpu.sync_copy(shared_scores, scores_v)

            def _key_chunk(c, acc):
                sl = pl.ds(c * _L, _L)
                iv = lax.bitcast_convert_type(scores_v[sl], jnp.int32)
                key = jnp.where(iv >= 0, iv, iv ^ jnp.int32(0x7FFFFFFF))
                keys_v[sl] = key
                return acc + jnp.where(key >= 0, 1, 0).astype(jnp.int32)

            cnt_pos = jnp.sum(
                lax.fori_loop(0, nvec, _key_chunk,
                              jnp.zeros((_L,), jnp.int32)))

            def count_ge(thresh):
                acc = jnp.zeros((_L,), jnp.int32)
                for c in range(nvec):
                    kv = keys_v[pl.ds(c * _L, _L)]
                    acc = acc + jnp.where(kv >= thresh, 1, 0).astype(jnp.int32)
                return jnp.sum(acc)

            pos_bucket = cnt_pos >= M
            lo0 = jnp.where(pos_bucket, jnp.int32(0), jnp.int32(-(2**31)))
            hi0 = jnp.where(pos_bucket, jnp.int32(2**31 - 1), jnp.int32(-1))

            def bis(_, carry):
                lo, hi = carry
                mid = lo + 1 + ((hi - lo - 1) >> 1)
                ge = count_ge(mid) >= M
                return (jnp.where(ge, mid, lo), jnp.where(ge, hi, mid - 1))

            v_key, _ = lax.fori_loop(0, 31, bis, (lo0, hi0))

            acc_gt = jnp.zeros((_L,), jnp.int32)
            for c in range(nvec):
                kv = keys_v[pl.ds(c * _L, _L)]
                acc_gt = acc_gt + jnp.where(kv > v_key, 1, 0).astype(jnp.int32)
            need_eq = M - jnp.sum(acc_gt)

            for c in range(128 // _L):
                kept_v[pl.ds(c * _L, _L)] = jnp.zeros((_L,), jnp.int32)

            def _compact(c, carry):
                pos, eq_used = carry
                kv = keys_v[pl.ds(c * _L, _L)]
                idxs = lax.iota(jnp.int32, _L) + c * _L
                gt = kv > v_key
                eq = kv == v_key
                eq_i = eq.astype(jnp.int32)
                pref = plsc.cumsum(eq_i)
                ties_before = eq_used + pref - eq_i
                keep = gt | (eq & (ties_before < need_eq))
                plsc.store_compressed(kept_v.at[pl.ds(pos, _L)], idxs,
                                      mask=keep)
                return (pos + jnp.sum(keep.astype(jnp.int32)),
                        eq_used + jnp.sum(eq_i))

            lax.fori_loop(0, nvec, _compact, (jnp.int32(0), jnp.int32(0)))
            pltpu.sync_copy(kept_v.at[pl.ds(0, m_pad)], shared_idx)

        plsc.subcore_barrier()

        bpc = B // 2

        @pl.when(sid < bpc * n_ch)
        def _gather():
            b = cid * bpc + sid // n_ch
            ch = lax.rem(sid, n_ch)
            row_off = _mo8(jnp.where(ch == n_ch - 1, tail_off, ch * _L))
            pltpu.sync_copy(shared_idx.at[pl.ds(row_off, _L)], idx16_v)
            gidx = idx16_v[...] + b * N
            pltpu.async_copy(vt_hbm.at[gidx], rows_v, sem).wait()
            pltpu.sync_copy(rows_v, out_hbm.at[b, pl.ds(row_off, _L)])

    return k


def kernel(visual_tokens, attention_weights):
    B, N, D = visual_tokens.shape
    H = attention_weights.shape[1]
    cls_rows = attention_weights[:, :, 0, 1:].reshape(B * H * N)
    vt2 = visual_tokens.reshape(B * N, D)
    return _build(B, N, D, H)(cls_rows, vt2)


# speedup vs baseline: 1.3418x; 1.0420x over previous
import functools

import jax
import jax.numpy as jnp
from jax import lax
from jax.experimental import pallas as pl
from jax.experimental.pallas import tpu as pltpu
from jax.experimental.pallas import tpu_sc as plsc

_REDUCTION_RATE = 0.9
_L = 16


def _mo8(x):
    return pl.multiple_of(x, 8)


def _build(B, N, D, H):
    M = max(1, int(N * (1.0 - _REDUCTION_RATE)))
    n_ch = -(-M // _L)
    m_pad = -(-M // 8) * 8
    tail_off = m_pad - _L
    rows_total = B * H
    rows_per_tile = rows_total // _L
    nvec = N // _L
    cols_per_tile = 128
    red_tiles = N // cols_per_tile
    mesh = plsc.VectorSubcoreMesh(core_axis_name="c", subcore_axis_name="s")

    @functools.partial(
        pl.kernel,
        out_type=jax.ShapeDtypeStruct((B, M, D), jnp.float32),
        mesh=mesh,
        scratch_types=[
            pltpu.VMEM((rows_per_tile, N), jnp.float32),
            pltpu.VMEM((N,), jnp.float32),
            pltpu.VMEM((_L, 128), jnp.float32),
            pltpu.VMEM((128,), jnp.float32),
            pltpu.VMEM((N,), jnp.float32),
            pltpu.VMEM((N,), jnp.int32),
            pltpu.VMEM((128,), jnp.int32),
            pltpu.VMEM((_L,), jnp.int32),
            pltpu.VMEM((_L, D), jnp.float32),
            pltpu.VMEM_SHARED((_L, N), jnp.float32),
            pltpu.VMEM_SHARED((N,), jnp.float32),
            pltpu.VMEM_SHARED((m_pad,), jnp.int32),
            pltpu.SemaphoreType.DMA,
        ],
        compiler_params=pltpu.CompilerParams(needs_layout_passes=False),
    )
    def k(cls_hbm, vt_hbm, out_hbm, row4_v, part_v, sub_v, loc_v, scores_v,
          keys_v, kept_v, idx16_v, rows_v, shared_part, shared_scores,
          shared_idx, sem):
        cid = lax.axis_index("c")
        sid = lax.axis_index("s")

        copies = []
        for j in range(rows_per_tile):
            r = sid * rows_per_tile + j
            copies.append(
                pltpu.async_copy(cls_hbm.at[pl.ds(_mo8(r * N), N)],
                                 row4_v.at[j], sem))
        for c in copies:
            c.wait()

        def _sum_chunk(c, _):
            sl = pl.ds(c * _L, _L)
            acc = row4_v[0, sl]
            for j in range(1, rows_per_tile):
                acc = acc + row4_v[j, sl]
            part_v[sl] = acc
            return 0

        lax.fori_loop(0, nvec, _sum_chunk, 0)
        pltpu.sync_copy(part_v, shared_part.at[sid])
        plsc.subcore_barrier()

        @pl.when(sid < red_tiles)
        def _reduce():
            col0 = pl.multiple_of(sid * cols_per_tile, cols_per_tile)
            pltpu.sync_copy(shared_part.at[:, pl.ds(col0, cols_per_tile)],
                            sub_v)
            for c in range(cols_per_tile // _L):
                sl = pl.ds(c * _L, _L)
                acc = sub_v[0, sl]
                for t in range(1, _L):
                    acc = acc + sub_v[t, sl]
                loc_v[sl] = acc
            pltpu.sync_copy(loc_v,
                            shared_scores.at[pl.ds(_mo8(col0),
                                                   cols_per_tile)])

        plsc.subcore_barrier()

        @pl.when(sid == 0)
        def _select():
            pltpu.sync_copy(shared_scores, scores_v)

            def _key_chunk(c, acc):
                sl = pl.ds(c * _L, _L)
                iv = lax.bitcast_convert_type(scores_v[sl], jnp.int32)
                key = jnp.where(iv >= 0, iv, iv ^ jnp.int32(0x7FFFFFFF))
                keys_v[sl] = key
                return acc + jnp.where(key >= 0, 1, 0).astype(jnp.int32)

            cnt_pos = jnp.sum(
                lax.fori_loop(0, nvec, _key_chunk,
                              jnp.zeros((_L,), jnp.int32)))

            def count_ge(thresh):
                acc = jnp.zeros((_L,), jnp.int32)
                for c in range(nvec):
                    kv = keys_v[pl.ds(c * _L, _L)]
                    acc = acc + jnp.where(kv >= thresh, 1, 0).astype(jnp.int32)
                return jnp.sum(acc)

            pos_bucket = cnt_pos >= M
            lo0 = jnp.where(pos_bucket, jnp.int32(0), jnp.int32(-(2**31)))
            hi0 = jnp.where(pos_bucket, jnp.int32(2**31 - 1), jnp.int32(-1))

            def bis(_, carry):
                lo, hi = carry
                mid = lo + 1 + ((hi - lo - 1) >> 1)
                ge = count_ge(mid) >= M
                return (jnp.where(ge, mid, lo), jnp.where(ge, hi, mid - 1))

            v_key, _ = lax.fori_loop(0, 31, bis, (lo0, hi0))

            acc_gt = jnp.zeros((_L,), jnp.int32)
            for c in range(nvec):
                kv = keys_v[pl.ds(c * _L, _L)]
                acc_gt = acc_gt + jnp.where(kv > v_key, 1, 0).astype(jnp.int32)
            need_eq = M - jnp.sum(acc_gt)

            for c in range(128 // _L):
                kept_v[pl.ds(c * _L, _L)] = jnp.zeros((_L,), jnp.int32)

            def _compact(c, carry):
                pos, eq_used = carry
                kv = keys_v[pl.ds(c * _L, _L)]
                idxs = lax.iota(jnp.int32, _L) + c * _L
                gt = kv > v_key
                eq = kv == v_key
                eq_i = eq.astype(jnp.int32)
                pref = plsc.cumsum(eq_i)
                ties_before = eq_used + pref - eq_i
                keep = gt | (eq & (ties_before < need_eq))
                plsc.store_compressed(kept_v.at[pl.ds(pos, _L)], idxs,
                                      mask=keep)
                return (pos + jnp.sum(keep.astype(jnp.int32)),
                        eq_used + jnp.sum(eq_i))

            lax.fori_loop(0, nvec, _compact, (jnp.int32(0), jnp.int32(0)))
            pltpu.sync_copy(kept_v.at[pl.ds(0, m_pad)], shared_idx)

        plsc.subcore_barrier()

        bpc = B // 2

        @pl.when(sid < bpc * n_ch)
        def _gather():
            b = cid * bpc + sid // n_ch
            ch = lax.rem(sid, n_ch)
            row_off = _mo8(jnp.where(ch == n_ch - 1, tail_off, ch * _L))
            pltpu.sync_copy(shared_idx.at[pl.ds(row_off, _L)], idx16_v)
            gidx = idx16_v[...] + b * N
            pltpu.async_copy(vt_hbm.at[gidx], rows_v, sem).wait()
            pltpu.sync_copy(rows_v, out_hbm.at[b, pl.ds(row_off, _L)])

    return k


def kernel(visual_tokens, attention_weights):
    B, N, D = visual_tokens.shape
    H = attention_weights.shape[1]
    cls_rows = attention_weights[:, :, 0, 1:].reshape(B * H * N)
    vt2 = visual_tokens.reshape(B * N, D)
    return _build(B, N, D, H)(cls_rows, vt2)


# speedup vs baseline: 1.3535x; 1.0087x over previous
import functools

import jax
import jax.numpy as jnp
from jax import lax
from jax.experimental import pallas as pl
from jax.experimental.pallas import tpu as pltpu
from jax.experimental.pallas import tpu_sc as plsc

_REDUCTION_RATE = 0.9
_L = 16


def _mo8(x):
    return pl.multiple_of(x, 8)


def _build(B, N, D, H):
    M = max(1, int(N * (1.0 - _REDUCTION_RATE)))
    n_ch = -(-M // _L)
    m_pad = -(-M // 8) * 8
    tail_off = m_pad - _L
    rows_total = B * H
    rows_per_tile = rows_total // _L
    nvec = N // _L
    cols_per_tile = 128
    red_tiles = N // cols_per_tile
    mesh = plsc.VectorSubcoreMesh(core_axis_name="c", subcore_axis_name="s")

    @functools.partial(
        pl.kernel,
        out_type=jax.ShapeDtypeStruct((B, M, D), jnp.float32),
        mesh=mesh,
        scratch_types=[
            pltpu.VMEM((rows_per_tile, N), jnp.float32),
            pltpu.VMEM((N,), jnp.float32),
            pltpu.VMEM((_L, 128), jnp.float32),
            pltpu.VMEM((128,), jnp.float32),
            pltpu.VMEM((N,), jnp.float32),
            pltpu.VMEM((N,), jnp.int32),
            pltpu.VMEM((128,), jnp.int32),
            pltpu.VMEM((_L,), jnp.int32),
            pltpu.VMEM((_L, D), jnp.float32),
            pltpu.VMEM_SHARED((_L, N), jnp.float32),
            pltpu.VMEM_SHARED((N,), jnp.float32),
            pltpu.VMEM_SHARED((m_pad,), jnp.int32),
            pltpu.SemaphoreType.DMA,
        ],
        compiler_params=pltpu.CompilerParams(needs_layout_passes=False),
    )
    def k(cls_hbm, vt_hbm, out_hbm, row4_v, part_v, sub_v, loc_v, scores_v,
          keys_v, kept_v, idx16_v, rows_v, shared_part, shared_scores,
          shared_idx, sem):
        cid = lax.axis_index("c")
        sid = lax.axis_index("s")

        copies = []
        for j in range(rows_per_tile):
            r = sid * rows_per_tile + j
            copies.append(
                pltpu.async_copy(cls_hbm.at[pl.ds(_mo8(r * N), N)],
                                 row4_v.at[j], sem))
        for c in copies:
            c.wait()

        def _sum_chunk(c, _):
            sl = pl.ds(c * _L, _L)
            acc = row4_v[0, sl]
            for j in range(1, rows_per_tile):
                acc = acc + row4_v[j, sl]
            part_v[sl] = acc
            return 0

        lax.fori_loop(0, nvec, _sum_chunk, 0)
        pltpu.sync_copy(part_v, shared_part.at[sid])
        plsc.subcore_barrier()

        @pl.when(sid < red_tiles)
        def _reduce():
            col0 = pl.multiple_of(sid * cols_per_tile, cols_per_tile)
            pltpu.sync_copy(shared_part.at[:, pl.ds(col0, cols_per_tile)],
                            sub_v)
            def _red_chunk(c, _):
                sl = pl.ds(c * _L, _L)
                acc = sub_v[0, sl]
                for t in range(1, _L):
                    acc = acc + sub_v[t, sl]
                loc_v[sl] = acc
                return 0

            lax.fori_loop(0, cols_per_tile // _L, _red_chunk, 0)
            pltpu.sync_copy(loc_v,
                            shared_scores.at[pl.ds(_mo8(col0),
                                                   cols_per_tile)])

        plsc.subcore_barrier()

        @pl.when(sid == 0)
        def _select():
            pltpu.sync_copy(shared_scores, scores_v)

            def _key_chunk(c, acc):
                sl = pl.ds(c * _L, _L)
                iv = lax.bitcast_convert_type(scores_v[sl], jnp.int32)
                key = jnp.where(iv >= 0, iv, iv ^ jnp.int32(0x7FFFFFFF))
                keys_v[sl] = key
                return acc + jnp.where(key >= 0, 1, 0).astype(jnp.int32)

            cnt_pos = jnp.sum(
                lax.fori_loop(0, nvec, _key_chunk,
                              jnp.zeros((_L,), jnp.int32)))

            def count_ge(thresh):
                acc = jnp.zeros((_L,), jnp.int32)
                for c in range(nvec):
                    kv = keys_v[pl.ds(c * _L, _L)]
                    acc = acc + jnp.where(kv >= thresh, 1, 0).astype(jnp.int32)
                return jnp.sum(acc)

            pos_bucket = cnt_pos >= M
            lo0 = jnp.where(pos_bucket, jnp.int32(0), jnp.int32(-(2**31)))
            hi0 = jnp.where(pos_bucket, jnp.int32(2**31 - 1), jnp.int32(-1))

            def bis(_, carry):
                lo, hi = carry
                mid = lo + 1 + ((hi - lo - 1) >> 1)
                ge = count_ge(mid) >= M
                return (jnp.where(ge, mid, lo), jnp.where(ge, hi, mid - 1))

            v_key, _ = lax.fori_loop(0, 31, bis, (lo0, hi0))

            def _gt_chunk(c, acc):
                kv = keys_v[pl.ds(c * _L, _L)]
                return acc + jnp.where(kv > v_key, 1, 0).astype(jnp.int32)

            need_eq = M - jnp.sum(
                lax.fori_loop(0, nvec, _gt_chunk,
                              jnp.zeros((_L,), jnp.int32)))

            def _zero_chunk(c, _):
                kept_v[pl.ds(c * _L, _L)] = jnp.zeros((_L,), jnp.int32)
                return 0

            lax.fori_loop(0, 128 // _L, _zero_chunk, 0)

            def _compact(c, carry):
                pos, eq_used = carry
                kv = keys_v[pl.ds(c * _L, _L)]
                idxs = lax.iota(jnp.int32, _L) + c * _L
                gt = kv > v_key
                eq = kv == v_key
                eq_i = eq.astype(jnp.int32)
                pref = plsc.cumsum(eq_i)
                ties_before = eq_used + pref - eq_i
                keep = gt | (eq & (ties_before < need_eq))
                plsc.store_compressed(kept_v.at[pl.ds(pos, _L)], idxs,
                                      mask=keep)
                return (pos + jnp.sum(keep.astype(jnp.int32)),
                        eq_used + jnp.sum(eq_i))

            lax.fori_loop(0, nvec, _compact, (jnp.int32(0), jnp.int32(0)))
            pltpu.sync_copy(kept_v.at[pl.ds(0, m_pad)], shared_idx)

        plsc.subcore_barrier()

        bpc = B // 2

        @pl.when(sid < bpc * n_ch)
        def _gather():
            b = cid * bpc + sid // n_ch
            ch = lax.rem(sid, n_ch)
            row_off = _mo8(jnp.where(ch == n_ch - 1, tail_off, ch * _L))
            pltpu.sync_copy(shared_idx.at[pl.ds(row_off, _L)], idx16_v)
            gidx = idx16_v[...] + b * N
            pltpu.async_copy(vt_hbm.at[gidx], rows_v, sem).wait()
            pltpu.sync_copy(rows_v, out_hbm.at[b, pl.ds(row_off, _L)])

    return k


def kernel(visual_tokens, attention_weights):
    B, N, D = visual_tokens.shape
    H = attention_weights.shape[1]
    cls_rows = attention_weights[:, :, 0, 1:].reshape(B * H * N)
    vt2 = visual_tokens.reshape(B * N, D)
    return _build(B, N, D, H)(cls_rows, vt2)
